# baseline (speedup 1.0000x reference)
import functools

import jax
import jax.numpy as jnp
from jax import lax
from jax.experimental import pallas as pl
from jax.experimental.pallas import tpu as pltpu
from jax.experimental.pallas import tpu_sc as plsc

F32 = jnp.float32
PREC = lax.Precision.HIGHEST
G = 64
BN = 1000
NC, NS = 2, 16
CH = 128


def _dot(a, b):
    return jnp.dot(a, b, preferred_element_type=F32, precision=PREC)


@functools.lru_cache(maxsize=None)
def _make_edge_pass(n_rows, width, ew, np_rows):
    mesh = plsc.VectorSubcoreMesh(core_axis_name="c", subcore_axis_name="s")
    n_super = ew // (CH * 2)
    rpt = np_rows // NS

    @functools.partial(
        pl.kernel,
        out_type=jax.ShapeDtypeStruct((NC, np_rows, width), F32),
        mesh=mesh,
        scratch_types=[
            pltpu.VMEM((CH,), jnp.int32),
            pltpu.VMEM((CH,), jnp.int32),
            pltpu.VMEM((CH,), jnp.int32),
            pltpu.VMEM((CH,), jnp.int32),
            pltpu.VMEM((CH, width), F32),
            pltpu.VMEM((CH, width), F32),
            pltpu.VMEM_SHARED((np_rows, width), F32),
            pltpu.SemaphoreType.DMA,
            pltpu.SemaphoreType.DMA,
        ],
    )
    def edge_pass(table_hbm, src_hbm, dst_hbm, zeros_hbm, out_hbm,
                  sidx_a, sidx_b, didx_a, didx_b, rows_a, rows_b,
                  acc_sh, sem_a, sem_b):
        c = lax.axis_index("c")
        s = lax.axis_index("s")
        wid = s * NC + c
        pltpu.sync_copy(zeros_hbm, acc_sh.at[pl.ds(s * rpt, rpt)])
        plsc.subcore_barrier()
        ebase = wid * ew

        def body(t, carry):
            off = ebase + t * (2 * CH)
            pltpu.sync_copy(src_hbm.at[pl.ds(off, CH)], sidx_a)
            pltpu.sync_copy(dst_hbm.at[pl.ds(off, CH)], didx_a)
            pltpu.async_copy(table_hbm.at[sidx_a], rows_a, sem_a).wait()
            pltpu.sync_copy(rows_a, acc_sh.at[didx_a], add=True)
            pltpu.sync_copy(src_hbm.at[pl.ds(off + CH, CH)], sidx_b)
            pltpu.sync_copy(dst_hbm.at[pl.ds(off + CH, CH)], didx_b)
            pltpu.async_copy(table_hbm.at[sidx_b], rows_b, sem_b).wait()
            pltpu.sync_copy(rows_b, acc_sh.at[didx_b], add=True)
            return carry

        lax.fori_loop(0, n_super, body, 0)
        plsc.subcore_barrier()
        pltpu.sync_copy(acc_sh.at[pl.ds(s * rpt, rpt)],
                        out_hbm.at[c, pl.ds(s * rpt, rpt)])

    return edge_pass


def _proj(x, w, b, do_relu):
    n, d = x.shape

    def body(x_ref, w_ref, b_ref, o_ref):
        y = _dot(x_ref[...], w_ref[...]) + b_ref[...]
        if do_relu:
            y = jnp.maximum(y, 0.0)
        o_ref[...] = y

    return pl.pallas_call(
        body,
        grid=(n // BN,),
        in_specs=[pl.BlockSpec((BN, d), lambda i: (i, 0)),
                  pl.BlockSpec((d, d), lambda i: (0, 0)),
                  pl.BlockSpec((1, d), lambda i: (0, 0))],
        out_specs=pl.BlockSpec((BN, d), lambda i: (i, 0)),
        out_shape=jax.ShapeDtypeStruct((n, d), F32),
    )(x, w, b.reshape(1, d))


def _layer_a(h, batch2d, vn, wm, ws):
    n, d = h.shape

    def body(h_ref, b_ref, vn_ref, wm_ref, ws_ref, hm_ref, hs_ref):
        oh = (b_ref[...] ==
              lax.broadcasted_iota(jnp.int32, (BN, G), 1)).astype(F32)
        h_in = h_ref[...] + _dot(oh, vn_ref[...])
        hm_ref[...] = _dot(h_in, wm_ref[...])
        hs_ref[...] = _dot(h_in, ws_ref[...])

    return pl.pallas_call(
        body,
        grid=(n // BN,),
        in_specs=[pl.BlockSpec((BN, d), lambda i: (i, 0)),
                  pl.BlockSpec((BN, 1), lambda i: (i, 0)),
                  pl.BlockSpec((G, d), lambda i: (0, 0)),
                  pl.BlockSpec((d, d), lambda i: (0, 0)),
                  pl.BlockSpec((d, d), lambda i: (0, 0))],
        out_specs=[pl.BlockSpec((BN, d), lambda i: (i, 0)),
                   pl.BlockSpec((BN, d), lambda i: (i, 0))],
        out_shape=[jax.ShapeDtypeStruct((n, d), F32),
                   jax.ShapeDtypeStruct((n, d), F32)],
    )(h, batch2d, vn, wm, ws)


def _layer_b(hs, agg, cnt, emb, bg, lns, lnb, h_prev):
    n, d = hs.shape
    wc = cnt.shape[2]

    def body(hs_ref, ag_ref, c_ref, e_ref, bg_ref, s_ref, b_ref, hp_ref,
             o_ref):
        a = hs_ref[...] + ag_ref[0] + ag_ref[1] + bg_ref[...]
        a = a + _dot(c_ref[0] + c_ref[1], e_ref[...])
        mu = jnp.mean(a, axis=-1, keepdims=True)
        t = a - mu
        var = jnp.mean(t * t, axis=-1, keepdims=True)
        o_ref[...] = (t * lax.rsqrt(var + 1e-5) * s_ref[...] + b_ref[...]
                      + hp_ref[...])

    return pl.pallas_call(
        body,
        grid=(n // BN,),
        in_specs=[pl.BlockSpec((BN, d), lambda i: (i, 0)),
                  pl.BlockSpec((NC, BN, d), lambda i: (0, i, 0)),
                  pl.BlockSpec((NC, BN, wc), lambda i: (0, i, 0)),
                  pl.BlockSpec((wc, d), lambda i: (0, 0)),
                  pl.BlockSpec((1, d), lambda i: (0, 0)),
                  pl.BlockSpec((1, d), lambda i: (0, 0)),
                  pl.BlockSpec((1, d), lambda i: (0, 0)),
                  pl.BlockSpec((BN, d), lambda i: (i, 0))],
        out_specs=pl.BlockSpec((BN, d), lambda i: (i, 0)),
        out_shape=jax.ShapeDtypeStruct((n, d), F32),
    )(hs, agg, cnt, emb, bg.reshape(1, d), lns.reshape(1, d),
      lnb.reshape(1, d), h_prev)


def _vn_update(h, batch2d, vn, w1, b1, w2, b2):
    n, d = h.shape
    nb = n // BN

    def body(h_ref, bt_ref, vn_ref, w1_ref, b1_ref, w2_ref, b2_ref, o_ref,
             acc_ref):
        i = pl.program_id(0)
        oh = (bt_ref[...] ==
              lax.broadcasted_iota(jnp.int32, (BN, G), 1)).astype(F32)
        part = lax.dot_general(oh, h_ref[...], (((0,), (0,)), ((), ())),
                               preferred_element_type=F32, precision=PREC)

        @pl.when(i == 0)
        def _():
            acc_ref[...] = part

        @pl.when(i > 0)
        def _():
            acc_ref[...] = acc_ref[...] + part

        @pl.when(i == nb - 1)
        def _():
            t = acc_ref[...] + vn_ref[...]
            t = jnp.maximum(_dot(t, w1_ref[...]) + b1_ref[...], 0.0)
            t = jnp.maximum(_dot(t, w2_ref[...]) + b2_ref[...], 0.0)
            o_ref[...] = vn_ref[...] + t

    return pl.pallas_call(
        body,
        grid=(nb,),
        in_specs=[pl.BlockSpec((BN, d), lambda i: (i, 0)),
                  pl.BlockSpec((BN, 1), lambda i: (i, 0)),
                  pl.BlockSpec((G, d), lambda i: (0, 0)),
                  pl.BlockSpec((d, d), lambda i: (0, 0)),
                  pl.BlockSpec((1, d), lambda i: (0, 0)),
                  pl.BlockSpec((d, d), lambda i: (0, 0)),
                  pl.BlockSpec((1, d), lambda i: (0, 0))],
        out_specs=pl.BlockSpec((G, d), lambda i: (0, 0)),
        out_shape=jax.ShapeDtypeStruct((G, d), F32),
        scratch_shapes=[pltpu.VMEM((G, d), F32)],
    )(h, batch2d, vn, w1, b1.reshape(1, d), w2, b2.reshape(1, d))


def kernel(x, W_init, b_init, Wmsg, Wself, bgnn, edge_emb, ln_s, ln_b, vn_emb,
           mlp_w1, mlp_b1, mlp_w2, mlp_b2, Wout, bout,
           edge_index, edge_attr, batch):
    n, d = x.shape
    n_layers = Wmsg.shape[0]
    e = edge_index.shape[1]
    ne = edge_emb.shape[1]

    nw = NC * NS
    ew = -(-e // (nw * CH * 2)) * CH * 2
    e_pad = nw * ew
    np_rows = -(-(n + 1) // (NS * 8)) * (NS * 8)
    rpt = np_rows // NS

    src = edge_index[0]
    dst = edge_index[1]
    pad = e_pad - e
    srcp = jnp.concatenate([src, jnp.zeros((pad,), jnp.int32)])
    dstp = jnp.concatenate([dst, jnp.full((pad,), n, jnp.int32)])
    attrp = jnp.concatenate([edge_attr, jnp.zeros((pad,), jnp.int32)])
    batch2d = batch.reshape(n, 1)
    zeros_d = jnp.zeros((rpt, d), F32)
    emb_pad = jnp.pad(edge_emb, ((0, 0), (0, d - ne), (0, 0)))

    edge_pass_d = _make_edge_pass(n, d, ew, np_rows)

    rep = 512
    onehot_tab = jnp.repeat(jnp.eye(16, d, dtype=F32), rep, axis=0)
    sidx_cnt = attrp * rep + (jnp.arange(e_pad, dtype=jnp.int32) & (rep - 1))
    cnt = _make_edge_pass(16 * rep, d, ew, np_rows)(
        onehot_tab, sidx_cnt, dstp, zeros_d)

    h = _proj(x, W_init, b_init, do_relu=False)
    vn = jnp.broadcast_to(vn_emb[0], (G, d))
    for l in range(n_layers):
        hm, hs = _layer_a(h, batch2d, vn, Wmsg[l], Wself[l])
        agg = edge_pass_d(hm, srcp, dstp, zeros_d)
        if l < n_layers - 1:
            vn = _vn_update(h, batch2d, vn, mlp_w1[l], mlp_b1[l],
                            mlp_w2[l], mlp_b2[l])
        h = _layer_b(hs, agg, cnt, emb_pad[l], bgnn[l], ln_s[l], ln_b[l], h)
    return _proj(h, Wout, bout, do_relu=True)


# speedup vs baseline: 1.3712x; 1.3712x over previous
import functools

import jax
import jax.numpy as jnp
from jax import lax
from jax.experimental import pallas as pl
from jax.experimental.pallas import tpu as pltpu
from jax.experimental.pallas import tpu_sc as plsc

F32 = jnp.float32
PREC = lax.Precision.HIGHEST
G = 64
BN = 1000
NC, NS = 2, 16
CH = 128


def _dot(a, b):
    return jnp.dot(a, b, preferred_element_type=F32, precision=PREC)


@functools.lru_cache(maxsize=None)
def _make_edge_pass(n_rows, width, ew, np_rows):
    mesh = plsc.VectorSubcoreMesh(core_axis_name="c", subcore_axis_name="s")
    n_chunks = ew // CH
    rpt = np_rows // NS

    @functools.partial(
        pl.kernel,
        out_type=jax.ShapeDtypeStruct((NC, np_rows, width), F32),
        mesh=mesh,
        scratch_types=[
            pltpu.VMEM((CH,), jnp.int32),
            pltpu.VMEM((CH,), jnp.int32),
            pltpu.VMEM((CH, width), F32),
            pltpu.VMEM_SHARED((np_rows, width), F32),
            pltpu.SemaphoreType.DMA,
        ],
    )
    def edge_pass(table_hbm, src_hbm, dst_hbm, zeros_hbm, out_hbm,
                  sidx_v, didx_v, rows_v, acc_sh, sem):
        c = lax.axis_index("c")
        s = lax.axis_index("s")
        wid = s * NC + c
        pltpu.sync_copy(zeros_hbm, acc_sh.at[pl.ds(s * rpt, rpt)])
        plsc.subcore_barrier()
        ebase = wid * ew

        def body(i, carry):
            off = ebase + i * CH
            pltpu.sync_copy(src_hbm.at[pl.ds(off, CH)], sidx_v)
            pltpu.sync_copy(dst_hbm.at[pl.ds(off, CH)], didx_v)
            pltpu.async_copy(table_hbm.at[sidx_v], rows_v, sem).wait()
            pltpu.sync_copy(rows_v, acc_sh.at[didx_v], add=True)
            return carry

        lax.fori_loop(0, n_chunks, body, 0)
        plsc.subcore_barrier()
        pltpu.sync_copy(acc_sh.at[pl.ds(s * rpt, rpt)],
                        out_hbm.at[c, pl.ds(s * rpt, rpt)])

    return edge_pass


def _proj(x, w, b, do_relu):
    n, d = x.shape

    def body(x_ref, w_ref, b_ref, o_ref):
        y = _dot(x_ref[...], w_ref[...]) + b_ref[...]
        if do_relu:
            y = jnp.maximum(y, 0.0)
        o_ref[...] = y

    return pl.pallas_call(
        body,
        grid=(n // BN,),
        in_specs=[pl.BlockSpec((BN, d), lambda i: (i, 0)),
                  pl.BlockSpec((d, d), lambda i: (0, 0)),
                  pl.BlockSpec((1, d), lambda i: (0, 0))],
        out_specs=pl.BlockSpec((BN, d), lambda i: (i, 0)),
        out_shape=jax.ShapeDtypeStruct((n, d), F32),
    )(x, w, b.reshape(1, d))


def _layer_a(h, batch2d, vn, wm, ws):
    n, d = h.shape

    def body(h_ref, b_ref, vn_ref, wm_ref, ws_ref, hm_ref, hs_ref):
        oh = (b_ref[...] ==
              lax.broadcasted_iota(jnp.int32, (BN, G), 1)).astype(F32)
        h_in = h_ref[...] + _dot(oh, vn_ref[...])
        hm_ref[...] = _dot(h_in, wm_ref[...])
        hs_ref[...] = _dot(h_in, ws_ref[...])

    return pl.pallas_call(
        body,
        grid=(n // BN,),
        in_specs=[pl.BlockSpec((BN, d), lambda i: (i, 0)),
                  pl.BlockSpec((BN, 1), lambda i: (i, 0)),
                  pl.BlockSpec((G, d), lambda i: (0, 0)),
                  pl.BlockSpec((d, d), lambda i: (0, 0)),
                  pl.BlockSpec((d, d), lambda i: (0, 0))],
        out_specs=[pl.BlockSpec((BN, d), lambda i: (i, 0)),
                   pl.BlockSpec((BN, d), lambda i: (i, 0))],
        out_shape=[jax.ShapeDtypeStruct((n, d), F32),
                   jax.ShapeDtypeStruct((n, d), F32)],
    )(h, batch2d, vn, wm, ws)


def _layer_b(hs, agg, cnt, emb, bg, lns, lnb, h_prev):
    n, d = hs.shape
    wc = cnt.shape[2]

    def body(hs_ref, ag_ref, c_ref, e_ref, bg_ref, s_ref, b_ref, hp_ref,
             o_ref):
        a = hs_ref[...] + ag_ref[0] + ag_ref[1] + bg_ref[...]
        a = a + _dot(c_ref[0] + c_ref[1], e_ref[...])
        mu = jnp.mean(a, axis=-1, keepdims=True)
        t = a - mu
        var = jnp.mean(t * t, axis=-1, keepdims=True)
        o_ref[...] = (t * lax.rsqrt(var + 1e-5) * s_ref[...] + b_ref[...]
                      + hp_ref[...])

    return pl.pallas_call(
        body,
        grid=(n // BN,),
        in_specs=[pl.BlockSpec((BN, d), lambda i: (i, 0)),
                  pl.BlockSpec((NC, BN, d), lambda i: (0, i, 0)),
                  pl.BlockSpec((NC, BN, wc), lambda i: (0, i, 0)),
                  pl.BlockSpec((wc, d), lambda i: (0, 0)),
                  pl.BlockSpec((1, d), lambda i: (0, 0)),
                  pl.BlockSpec((1, d), lambda i: (0, 0)),
                  pl.BlockSpec((1, d), lambda i: (0, 0)),
                  pl.BlockSpec((BN, d), lambda i: (i, 0))],
        out_specs=pl.BlockSpec((BN, d), lambda i: (i, 0)),
        out_shape=jax.ShapeDtypeStruct((n, d), F32),
    )(hs, agg, cnt, emb, bg.reshape(1, d), lns.reshape(1, d),
      lnb.reshape(1, d), h_prev)


def _vn_update(h, batch2d, vn, w1, b1, w2, b2):
    n, d = h.shape
    nb = n // BN

    def body(h_ref, bt_ref, vn_ref, w1_ref, b1_ref, w2_ref, b2_ref, o_ref,
             acc_ref):
        i = pl.program_id(0)
        oh = (bt_ref[...] ==
              lax.broadcasted_iota(jnp.int32, (BN, G), 1)).astype(F32)
        part = lax.dot_general(oh, h_ref[...], (((0,), (0,)), ((), ())),
                               preferred_element_type=F32, precision=PREC)

        @pl.when(i == 0)
        def _():
            acc_ref[...] = part

        @pl.when(i > 0)
        def _():
            acc_ref[...] = acc_ref[...] + part

        @pl.when(i == nb - 1)
        def _():
            t = acc_ref[...] + vn_ref[...]
            t = jnp.maximum(_dot(t, w1_ref[...]) + b1_ref[...], 0.0)
            t = jnp.maximum(_dot(t, w2_ref[...]) + b2_ref[...], 0.0)
            o_ref[...] = vn_ref[...] + t

    return pl.pallas_call(
        body,
        grid=(nb,),
        in_specs=[pl.BlockSpec((BN, d), lambda i: (i, 0)),
                  pl.BlockSpec((BN, 1), lambda i: (i, 0)),
                  pl.BlockSpec((G, d), lambda i: (0, 0)),
                  pl.BlockSpec((d, d), lambda i: (0, 0)),
                  pl.BlockSpec((1, d), lambda i: (0, 0)),
                  pl.BlockSpec((d, d), lambda i: (0, 0)),
                  pl.BlockSpec((1, d), lambda i: (0, 0))],
        out_specs=pl.BlockSpec((G, d), lambda i: (0, 0)),
        out_shape=jax.ShapeDtypeStruct((G, d), F32),
        scratch_shapes=[pltpu.VMEM((G, d), F32)],
    )(h, batch2d, vn, w1, b1.reshape(1, d), w2, b2.reshape(1, d))


def kernel(x, W_init, b_init, Wmsg, Wself, bgnn, edge_emb, ln_s, ln_b, vn_emb,
           mlp_w1, mlp_b1, mlp_w2, mlp_b2, Wout, bout,
           edge_index, edge_attr, batch):
    n, d = x.shape
    n_layers = Wmsg.shape[0]
    e = edge_index.shape[1]
    ne = edge_emb.shape[1]

    nw = NC * NS
    ew = -(-e // (nw * CH)) * CH
    e_pad = nw * ew
    np_rows = -(-(n + 1) // (NS * 8)) * (NS * 8)
    rpt = np_rows // NS

    src = edge_index[0]
    dst = edge_index[1]
    pad = e_pad - e
    srcp = jnp.concatenate([src, jnp.zeros((pad,), jnp.int32)])
    dstp = jnp.concatenate([dst, jnp.full((pad,), n, jnp.int32)])
    attrp = jnp.concatenate([edge_attr, jnp.zeros((pad,), jnp.int32)])
    batch2d = batch.reshape(n, 1)
    zeros_d = jnp.zeros((rpt, d), F32)
    emb_pad = jnp.pad(edge_emb, ((0, 0), (0, d - ne), (0, 0)))

    edge_pass_d = _make_edge_pass(n, d, ew, np_rows)

    rep = 512
    onehot_tab = jnp.repeat(jnp.eye(16, d, dtype=F32), rep, axis=0)
    sidx_cnt = attrp * rep + (jnp.arange(e_pad, dtype=jnp.int32) & (rep - 1))
    cnt = _make_edge_pass(16 * rep, d, ew, np_rows)(
        onehot_tab, sidx_cnt, dstp, zeros_d)

    h = _proj(x, W_init, b_init, do_relu=False)
    vn = jnp.broadcast_to(vn_emb[0], (G, d))
    for l in range(n_layers):
        hm, hs = _layer_a(h, batch2d, vn, Wmsg[l], Wself[l])
        agg = edge_pass_d(hm, srcp, dstp, zeros_d)
        if l < n_layers - 1:
            vn = _vn_update(h, batch2d, vn, mlp_w1[l], mlp_b1[l],
                            mlp_w2[l], mlp_b2[l])
        h = _layer_b(hs, agg, cnt, emb_pad[l], bgnn[l], ln_s[l], ln_b[l], h)
    return _proj(h, Wout, bout, do_relu=True)
